# speedup vs baseline: 1.9993x; 1.9993x over previous
import jax
import jax.numpy as jnp
from jax.experimental import pallas as pl

TOK = 2048
D = 1024
G = 8
S = 2
NF = 64
FPG = 8
DF = 64
DR = 128
DH = 256
GROUP_TOP_K = 4
TEMP = 1.0

TILE = 256


def _moe_kernel(hidden_ref, feat_ref, lng_ref, lnb_ref,
                wp_ref, bp_ref, wr1h_ref, wr1f_ref, br1_ref, wr2_ref, br2_ref,
                we_ref, be_ref, w1_ref, b1_ref, w2_ref, b2_ref,
                out_ref):
    x = hidden_ref[...]

    mu = jnp.mean(x, axis=-1, keepdims=True)
    xc = x - mu
    var = jnp.mean(xc * xc, axis=-1, keepdims=True)
    h = xc * jax.lax.rsqrt(var + 1e-5) * lng_ref[...] + lnb_ref[...]
    hb = h.astype(jnp.bfloat16)

    femb = jnp.dot(feat_ref[...].astype(jnp.bfloat16), wp_ref[...],
                   preferred_element_type=jnp.float32)
    femb = femb.reshape(TILE, G, DF) + bp_ref[...][None]

    glog_cols = []
    for g in range(G):
        rh = jnp.dot(hb, wr1h_ref[g], preferred_element_type=jnp.float32)
        rh += jnp.dot(femb[:, g].astype(jnp.bfloat16), wr1f_ref[g],
                      preferred_element_type=jnp.float32)
        rh = jax.nn.gelu(rh + br1_ref[g][None])
        glog_cols.append(jnp.sum(rh * wr2_ref[g][None], axis=-1, keepdims=True)
                         + br2_ref[g])
    glogits = jnp.concatenate(glog_cols, axis=-1) / max(TEMP, 1e-6)

    work = glogits
    neg = jnp.float32(-jnp.inf)
    thr = None
    for _ in range(GROUP_TOP_K):
        thr = jnp.max(work, axis=-1, keepdims=True)
        work = jnp.where(work >= thr, neg, work)
    keep = glogits >= thr
    gmax = jnp.max(glogits, axis=-1, keepdims=True)
    ge = jnp.where(keep, jnp.exp(glogits - gmax), 0.0)
    gw = ge / jnp.sum(ge, axis=-1, keepdims=True)

    elogits = (jnp.dot(hb, we_ref[...], preferred_element_type=jnp.float32)
               + be_ref[...]) / max(TEMP, 1e-6)
    el = elogits.reshape(TILE, G, S)
    em = jnp.max(el, axis=-1, keepdims=True)
    ee = jnp.exp(el - em)
    ew = ee / jnp.sum(ee, axis=-1, keepdims=True)

    cw = (gw[:, :, None] * ew).reshape(TILE, G * S)

    acc = x
    for e in range(G * S):
        u = jnp.dot(hb, w1_ref[e], preferred_element_type=jnp.float32)
        u = jax.nn.gelu(u + b1_ref[e][None]).astype(jnp.bfloat16)
        v = jnp.dot(u, w2_ref[e], preferred_element_type=jnp.float32)
        acc = acc + cw[:, e][:, None] * (v + b2_ref[e][None])
    out_ref[...] = acc


@jax.jit
def kernel(hidden, features, ln_g, ln_b, Wp, bp, Wr1, br1, Wr2, br2,
           We, be, W1, b1, W2, b2, group_idx):
    B = hidden.shape[0]

    onehot = jax.nn.one_hot(group_idx, NF, dtype=Wp.dtype, axis=0)
    wp_full = jnp.einsum('ngf,gfd->ngd', onehot, Wp).reshape(NF, G * DF)
    wp_full = wp_full.astype(jnp.bfloat16)

    wr1h = Wr1[:, :D, :].astype(jnp.bfloat16)
    wr1f = Wr1[:, D:, :].astype(jnp.bfloat16)
    wr2 = Wr2[..., 0]
    we_flat = jnp.transpose(We, (1, 0, 2)).reshape(D, G * S)
    we_flat = we_flat.astype(jnp.bfloat16)
    be_flat = be.reshape(1, G * S)
    w1 = W1.astype(jnp.bfloat16)
    w2 = W2.astype(jnp.bfloat16)

    n_tiles = B // TILE
    full = lambda shape: pl.BlockSpec(shape, lambda i: (0,) * len(shape))

    out = pl.pallas_call(
        _moe_kernel,
        grid=(n_tiles,),
        in_specs=[
            pl.BlockSpec((TILE, D), lambda i: (i, 0)),
            pl.BlockSpec((TILE, NF), lambda i: (i, 0)),
            full((1, D)), full((1, D)),
            full((NF, G * DF)), full((G, DF)),
            full((G, D, DR)), full((G, DF, DR)), full((G, DR)),
            full((G, DR)), full((G, 1)),
            full((D, G * S)), full((1, G * S)),
            full((G * S, D, DH)), full((G * S, DH)),
            full((G * S, DH, D)), full((G * S, D)),
        ],
        out_specs=pl.BlockSpec((TILE, D), lambda i: (i, 0)),
        out_shape=jax.ShapeDtypeStruct((B, D), jnp.float32),
    )(hidden, features, ln_g.reshape(1, D), ln_b.reshape(1, D),
      wp_full, bp, wr1h, wr1f, br1, wr2, br2,
      we_flat, be_flat, w1, b1, w2, b2)
    return out
